# Initial kernel scaffold; baseline (speedup 1.0000x reference)
import jax
import jax.numpy as jnp
from jax.experimental import pallas as pl


def kernel(x, edge_index, edge_attr, We, be, emb_g, emb_b, W1, b1, g1, bb1, W2, b2, Wu, bu, gu, bbu, ln_g, ln_b):
    raise NotImplementedError("write your pallas kernel here")


# speedup vs baseline: 3.5015x; 3.5015x over previous
import functools

import jax
import jax.numpy as jnp
from jax import lax
from jax.experimental import pallas as pl
from jax.experimental.pallas import tpu as pltpu
from jax.experimental.pallas import tpu_sc as plsc

N = 10000
E = 160000
ND = 128
ED = 16
H = 128

NC = 2
NS = 16
NW = NC * NS
CH = 128
NCHUNK = E // CH
NPW = (NCHUNK + NW - 1) // NW


def _sc_mesh():
    return plsc.VectorSubcoreMesh(
        core_axis_name="c", subcore_axis_name="s", num_cores=NC, num_subcores=NS
    )


def _sc_gather(h, src, dst):
    @functools.partial(
        pl.kernel,
        out_type=(
            jax.ShapeDtypeStruct((E, H), jnp.float32),
            jax.ShapeDtypeStruct((E, H), jnp.float32),
        ),
        mesh=_sc_mesh(),
        scratch_types=[
            pltpu.VMEM((CH,), jnp.int32),
            pltpu.VMEM((CH,), jnp.int32),
            pltpu.VMEM((CH, H), jnp.float32),
            pltpu.VMEM((CH, H), jnp.float32),
            pltpu.SemaphoreType.DMA,
            pltpu.SemaphoreType.DMA,
        ],
    )
    def k(h_hbm, src_hbm, dst_hbm, xj_hbm, xi_hbm, sidx, didx, jrows, irows, sem_j, sem_i):
        wid = lax.axis_index("s") * NC + lax.axis_index("c")

        def body(t, carry):
            chunk = t * NW + wid

            @pl.when(chunk < NCHUNK)
            def _():
                base = chunk * CH
                pltpu.sync_copy(src_hbm.at[pl.ds(base, CH)], sidx)
                pltpu.sync_copy(dst_hbm.at[pl.ds(base, CH)], didx)
                cj = pltpu.async_copy(h_hbm.at[sidx], jrows, sem_j)
                ci = pltpu.async_copy(h_hbm.at[didx], irows, sem_i)
                cj.wait()
                ci.wait()
                pltpu.sync_copy(jrows, xj_hbm.at[pl.ds(base, CH)])
                pltpu.sync_copy(irows, xi_hbm.at[pl.ds(base, CH)])

            return carry

        lax.fori_loop(0, NPW, body, 0)

    return k(h, src, dst)


def _sc_scatter(m2, dst, zeros_nh):
    @functools.partial(
        pl.kernel,
        out_type=jax.ShapeDtypeStruct((NC, N, H), jnp.float32),
        mesh=_sc_mesh(),
        scratch_types=[
            pltpu.VMEM((CH,), jnp.int32),
            pltpu.VMEM((CH, H), jnp.float32),
            pltpu.VMEM_SHARED((N, H), jnp.float32),
        ],
    )
    def k(m2_hbm, dst_hbm, z_hbm, out_hbm, idx, rows, acc):
        cid = lax.axis_index("c")
        sid = lax.axis_index("s")
        wid = sid * NC + cid
        CZ = 400
        NZ = N // CZ

        for q in range((NZ + NS - 1) // NS):
            zc = q * NS + sid

            @pl.when(zc < NZ)
            def _():
                pltpu.sync_copy(
                    z_hbm.at[pl.ds(zc * CZ, CZ)],
                    acc.at[pl.ds(zc * CZ, CZ)],
                )

        plsc.subcore_barrier()

        def body(t, carry):
            chunk = t * NW + wid

            @pl.when(chunk < NCHUNK)
            def _():
                base = chunk * CH
                pltpu.sync_copy(dst_hbm.at[pl.ds(base, CH)], idx)
                pltpu.sync_copy(m2_hbm.at[pl.ds(base, CH)], rows)
                pltpu.sync_copy(rows, acc.at[idx], add=True)

            return carry

        lax.fori_loop(0, NPW, body, 0)
        plsc.subcore_barrier()
        for q in range((NZ + NS - 1) // NS):
            zc = q * NS + sid

            @pl.when(zc < NZ)
            def _():
                pltpu.sync_copy(
                    acc.at[pl.ds(zc * CZ, CZ)],
                    out_hbm.at[cid, pl.ds(zc * CZ, CZ)],
                )

    return k(m2, dst, zeros_nh)


def _tc_embed(x, We, b, s, t):
    BN_ = 1000

    def body(x_ref, w_ref, b_ref, s_ref, t_ref, o_ref):
        z = jnp.dot(x_ref[...], w_ref[...], preferred_element_type=jnp.float32)
        z = (z + b_ref[...]) * s_ref[...] + t_ref[...]
        o_ref[...] = jnp.maximum(z, 0.0)

    return pl.pallas_call(
        body,
        grid=(N // BN_,),
        in_specs=[
            pl.BlockSpec((BN_, ND), lambda i: (i, 0)),
            pl.BlockSpec((ND, H), lambda i: (0, 0)),
            pl.BlockSpec((1, H), lambda i: (0, 0)),
            pl.BlockSpec((1, H), lambda i: (0, 0)),
            pl.BlockSpec((1, H), lambda i: (0, 0)),
        ],
        out_specs=pl.BlockSpec((BN_, H), lambda i: (i, 0)),
        out_shape=jax.ShapeDtypeStruct((N, H), jnp.float32),
    )(x, We, b.reshape(1, H), s.reshape(1, H), t.reshape(1, H))


def _tc_edge(xi, xj, ea, W1i, W1j, W1e, b1, s1, t1, W2, b2):
    BE = 2000
    H2 = 2 * H

    def body(xi_ref, xj_ref, ea_ref, wi_ref, wj_ref, we_ref, b1_ref, s1_ref,
             t1_ref, w2_ref, b2_ref, o_ref):
        pre = jnp.dot(xi_ref[...], wi_ref[...], preferred_element_type=jnp.float32)
        pre += jnp.dot(xj_ref[...], wj_ref[...], preferred_element_type=jnp.float32)
        pre += jnp.dot(ea_ref[...], we_ref[...], preferred_element_type=jnp.float32)
        pre = (pre + b1_ref[...]) * s1_ref[...] + t1_ref[...]
        m = jnp.maximum(pre, 0.0)
        o_ref[...] = (
            jnp.dot(m, w2_ref[...], preferred_element_type=jnp.float32) + b2_ref[...]
        )

    return pl.pallas_call(
        body,
        grid=(E // BE,),
        in_specs=[
            pl.BlockSpec((BE, H), lambda i: (i, 0)),
            pl.BlockSpec((BE, H), lambda i: (i, 0)),
            pl.BlockSpec((BE, ED), lambda i: (i, 0)),
            pl.BlockSpec((H, H2), lambda i: (0, 0)),
            pl.BlockSpec((H, H2), lambda i: (0, 0)),
            pl.BlockSpec((ED, H2), lambda i: (0, 0)),
            pl.BlockSpec((1, H2), lambda i: (0, 0)),
            pl.BlockSpec((1, H2), lambda i: (0, 0)),
            pl.BlockSpec((1, H2), lambda i: (0, 0)),
            pl.BlockSpec((H2, H), lambda i: (0, 0)),
            pl.BlockSpec((1, H), lambda i: (0, 0)),
        ],
        out_specs=pl.BlockSpec((BE, H), lambda i: (i, 0)),
        out_shape=jax.ShapeDtypeStruct((E, H), jnp.float32),
    )(
        xi, xj, ea, W1i, W1j, W1e,
        b1.reshape(1, H2), s1.reshape(1, H2), t1.reshape(1, H2),
        W2, b2.reshape(1, H),
    )


def _tc_update(h, p0, p1, W1ij, b1, s1, t1, W2, b2, Wuh, Wua, bu, su, tu, lng, lnb):
    BN_ = 1000
    H2 = 2 * H

    def body(h_ref, p0_ref, p1_ref, wij_ref, b1_ref, s1_ref, t1_ref, w2_ref,
             b2_ref, wuh_ref, wua_ref, bu_ref, su_ref, tu_ref, g_ref, bb_ref,
             o_ref):
        hb = h_ref[...]
        pre = jnp.dot(hb, wij_ref[...], preferred_element_type=jnp.float32)
        pre = (pre + b1_ref[...]) * s1_ref[...] + t1_ref[...]
        mself = jnp.maximum(pre, 0.0)
        m2self = (
            jnp.dot(mself, w2_ref[...], preferred_element_type=jnp.float32)
            + b2_ref[...]
        )
        agg = p0_ref[...] + p1_ref[...] + m2self
        u = jnp.dot(hb, wuh_ref[...], preferred_element_type=jnp.float32)
        u += jnp.dot(agg, wua_ref[...], preferred_element_type=jnp.float32)
        u = (u + bu_ref[...]) * su_ref[...] + tu_ref[...]
        u = jnp.maximum(u, 0.0)
        mu = jnp.mean(u, axis=-1, keepdims=True)
        var = jnp.mean((u - mu) ** 2, axis=-1, keepdims=True)
        out = (u - mu) / jnp.sqrt(var + 1e-5) * g_ref[...] + bb_ref[...]
        o_ref[...] = hb + out

    return pl.pallas_call(
        body,
        grid=(N // BN_,),
        in_specs=[
            pl.BlockSpec((BN_, H), lambda i: (i, 0)),
            pl.BlockSpec((BN_, H), lambda i: (i, 0)),
            pl.BlockSpec((BN_, H), lambda i: (i, 0)),
            pl.BlockSpec((H, H2), lambda i: (0, 0)),
            pl.BlockSpec((1, H2), lambda i: (0, 0)),
            pl.BlockSpec((1, H2), lambda i: (0, 0)),
            pl.BlockSpec((1, H2), lambda i: (0, 0)),
            pl.BlockSpec((H2, H), lambda i: (0, 0)),
            pl.BlockSpec((1, H), lambda i: (0, 0)),
            pl.BlockSpec((H, H), lambda i: (0, 0)),
            pl.BlockSpec((H, H), lambda i: (0, 0)),
            pl.BlockSpec((1, H), lambda i: (0, 0)),
            pl.BlockSpec((1, H), lambda i: (0, 0)),
            pl.BlockSpec((1, H), lambda i: (0, 0)),
            pl.BlockSpec((1, H), lambda i: (0, 0)),
            pl.BlockSpec((1, H), lambda i: (0, 0)),
        ],
        out_specs=pl.BlockSpec((BN_, H), lambda i: (i, 0)),
        out_shape=jax.ShapeDtypeStruct((N, H), jnp.float32),
    )(
        h, p0, p1, W1ij,
        b1.reshape(1, H2), s1.reshape(1, H2), t1.reshape(1, H2),
        W2, b2.reshape(1, H), Wuh, Wua,
        bu.reshape(1, H), su.reshape(1, H), tu.reshape(1, H),
        lng.reshape(1, H), lnb.reshape(1, H),
    )


def kernel(x, edge_index, edge_attr, We, be, emb_g, emb_b, W1, b1, g1, bb1,
           W2, b2, Wu, bu, gu, bbu, ln_g, ln_b):
    inv = jnp.float32(1.0) / jnp.sqrt(jnp.float32(1.0 + 1e-5))
    src = edge_index[0]
    dst = edge_index[1]
    h = _tc_embed(x, We, be, emb_g * inv, emb_b)
    zeros_nh = jnp.zeros((N, H), jnp.float32)
    for l in range(W1.shape[0]):
        W1l = W1[l]
        W1i = W1l[:H]
        W1j = W1l[H:2 * H]
        W1e = W1l[2 * H:]
        s1 = g1[l] * inv
        xj, xi = _sc_gather(h, src, dst)
        m2 = _tc_edge(xi, xj, edge_attr, W1i, W1j, W1e, b1[l], s1, bb1[l],
                      W2[l], b2[l])
        parts = _sc_scatter(m2, dst, zeros_nh)
        h = _tc_update(h, parts[0], parts[1], W1i + W1j, b1[l], s1, bb1[l],
                       W2[l], b2[l], Wu[l][:H], Wu[l][H:], bu[l], gu[l] * inv,
                       bbu[l], ln_g[l], ln_b[l])
    return h
